# Initial kernel scaffold; baseline (speedup 1.0000x reference)
import jax
import jax.numpy as jnp
from jax.experimental import pallas as pl


def kernel(x, edge_index, W1, b1, W2, b2):
    raise NotImplementedError("write your pallas kernel here")


# speedup vs baseline: 35.5680x; 35.5680x over previous
import jax
import jax.numpy as jnp
from jax import lax
from jax.experimental import pallas as pl
from jax.experimental.pallas import tpu as pltpu
from jax.experimental.pallas import tpu_sc as plsc

NC = 2
NS = 16
CW = 128


def _build_sc_pass(n_rows, n_pad, ncols, use_table):
    SL = n_pad // NS
    rows_per_core = n_rows // NC

    scratch = []
    scratch += [pltpu.VMEM_SHARED((n_pad,), jnp.float32)] * ncols
    if use_table:
        scratch += [pltpu.VMEM_SHARED((n_pad,), jnp.float32)] * ncols
    scratch.append(pltpu.VMEM((CW,), jnp.int32))
    scratch.append(pltpu.VMEM((CW,), jnp.int32))
    scratch += [pltpu.VMEM((CW,), jnp.float32)] * ncols

    mesh = plsc.VectorSubcoreMesh(
        core_axis_name="c", subcore_axis_name="s",
        num_cores=NC, num_subcores=NS)

    n_in = 4 + (ncols if use_table else 0)

    def body(*refs):
        src_hbm, dst_hbm, zeros_hbm, ones_hbm = refs[0:4]
        tabs_hbm = refs[4:n_in]
        out_hbm = refs[n_in]
        i = n_in + 1
        accs = refs[i:i + ncols]; i += ncols
        tabs = refs[i:i + ncols] if use_table else ()
        i += ncols if use_table else 0
        src_v = refs[i]; dst_v = refs[i + 1]
        msgs = refs[i + 2:i + 2 + ncols]

        c = lax.axis_index("c")
        s = lax.axis_index("s")
        sl0 = pl.multiple_of(s * SL, 8)

        for k in range(ncols):
            pltpu.sync_copy(zeros_hbm.at[pl.ds(sl0, SL)],
                            accs[k].at[pl.ds(sl0, SL)])
            if use_table:
                pltpu.sync_copy(tabs_hbm[k].at[pl.ds(sl0, SL)],
                                tabs[k].at[pl.ds(sl0, SL)])
        if not use_table:
            pltpu.sync_copy(ones_hbm, msgs[0])
        plsc.subcore_barrier()

        base = c * rows_per_core
        lo = base + (s * rows_per_core) // NS
        hi = base + ((s + 1) * rows_per_core) // NS

        def step(j, carry):
            pltpu.sync_copy(dst_hbm.at[j], dst_v)
            if use_table:
                pltpu.sync_copy(src_hbm.at[j], src_v)
                for k in range(ncols):
                    pltpu.sync_copy(tabs[k].at[src_v], msgs[k])
            for k in range(ncols):
                pltpu.sync_copy(msgs[k], accs[k].at[dst_v], add=True)
            return carry

        lax.fori_loop(lo, hi, step, 0)
        plsc.subcore_barrier()

        for k in range(ncols):
            o0 = pl.multiple_of((c * ncols + k) * n_pad + sl0, 8)
            pltpu.sync_copy(accs[k].at[pl.ds(sl0, SL)],
                            out_hbm.at[pl.ds(o0, SL)])

    flat = pl.kernel(
        body,
        out_type=jax.ShapeDtypeStruct((NC * ncols * n_pad,), jnp.float32),
        mesh=mesh,
        scratch_types=scratch,
    )

    def run(*args):
        return flat(*args).reshape(NC, ncols, n_pad)

    return run


def _tc_call(body, n_out, rows):
    return pl.pallas_call(
        body,
        out_shape=[jax.ShapeDtypeStruct((rows, 128), jnp.float32)] * n_out,
    )


def kernel(x, edge_index, W1, b1, W2, b2):
    N = x.shape[0]
    E = edge_index.shape[1]
    fin = W1.shape[0]
    fmid = W1.shape[1]

    n_pad = ((N + 1023) // 1024) * 1024
    if n_pad == N:
        n_pad += 1024
    rows2d = n_pad // 128

    epad = ((E + CW * NC - 1) // (CW * NC)) * (CW * NC)
    if epad != E:
        fill = jnp.full((2, epad - E), n_pad - 1, dtype=edge_index.dtype)
        edge_index = jnp.concatenate([edge_index, fill], axis=1)
    R = epad // CW
    src2d = edge_index[0].reshape(R, CW)
    dst2d = edge_index[1].reshape(R, CW)

    zeros = jnp.zeros((n_pad,), jnp.float32)
    ones = jnp.ones((CW,), jnp.float32)

    xcols = jnp.zeros((fin, n_pad), jnp.float32).at[:, :N].set(x.T)

    deg_pass = _build_sc_pass(R, n_pad, 1, use_table=False)
    l1_pass = _build_sc_pass(R, n_pad, fin, use_table=True)
    l2_pass = _build_sc_pass(R, n_pad, 1, use_table=True)

    degp = deg_pass(src2d, dst2d, zeros, ones)

    def tc1(d0, d1, *refs):
        xr = refs[:fin]
        dinv_o = refs[fin]
        xs_o = refs[fin + 1:]
        deg = d0[...] + d1[...] + 1.0
        y = lax.rsqrt(deg)
        dinv = y * (1.5 - 0.5 * deg * y * y)
        dinv_o[...] = dinv
        for k in range(fin):
            xs_o[k][...] = xr[k][...] * dinv

    tc1_out = _tc_call(tc1, 1 + fin, rows2d)(
        degp[0, 0].reshape(rows2d, 128), degp[1, 0].reshape(rows2d, 128),
        *[xcols[k].reshape(rows2d, 128) for k in range(fin)])
    dinv2d = tc1_out[0]
    xs2d = tc1_out[1:]

    xs_flat = [r.reshape(n_pad) for r in xs2d]
    s1 = l1_pass(src2d, dst2d, zeros, ones, *xs_flat)

    def tc2(dinv_r, w1_r, b1_r, w2_r, *refs):
        xs_r = refs[:fin]
        s1_r = refs[fin:fin + 2 * fin]
        gs_o = refs[2 * fin + fin]
        dinv = dinv_r[...]
        agg = [dinv * (s1_r[2 * k][...] + s1_r[2 * k + 1][...] + xs_r[k][...])
               for k in range(fin)]
        g = jnp.zeros_like(dinv)
        for j in range(fmid):
            t = b1_r[0, j]
            for k in range(fin):
                t = t + agg[k] * w1_r[k, j]
            g = g + jnp.maximum(t, 0.0) * w2_r[0, j]
        gs_o[...] = g * dinv

    s1_2d = []
    for k in range(fin):
        s1_2d.append(s1[0, k].reshape(rows2d, 128))
        s1_2d.append(s1[1, k].reshape(rows2d, 128))
    gs2d = _tc_call(tc2, 1, rows2d)(
        dinv2d, W1, b1.reshape(1, fmid), W2.reshape(1, fmid),
        *[xs2d[k] for k in range(fin)], *s1_2d)[0]

    s2 = l2_pass(src2d, dst2d, zeros, ones, gs2d.reshape(n_pad))

    def tc3(dinv_r, gs_r, s20_r, s21_r, b2_r, out_o):
        out_o[...] = dinv_r[...] * (s20_r[...] + s21_r[...] + gs_r[...]) \
            + b2_r[0, 0]

    out2d = _tc_call(tc3, 1, rows2d)(
        dinv2d, gs2d, s2[0, 0].reshape(rows2d, 128),
        s2[1, 0].reshape(rows2d, 128), b2.reshape(1, 1))[0]

    return out2d.reshape(n_pad)[:N].reshape(N, 1)


# speedup vs baseline: 165.7693x; 4.6606x over previous
import jax
import jax.numpy as jnp
from jax import lax
from jax.experimental import pallas as pl
from jax.experimental.pallas import tpu as pltpu
from jax.experimental.pallas import tpu_sc as plsc

NC = 2
NS = 16
CW = 128


K = 8


def _build_sc_pass(n_rows, n_pad, ncols, use_table):
    SL = n_pad // NS
    rows_per_core = n_rows // NC

    scratch = []
    scratch += [pltpu.VMEM_SHARED((n_pad,), jnp.float32)] * ncols
    if use_table:
        scratch += [pltpu.VMEM_SHARED((n_pad,), jnp.float32)] * ncols
    scratch.append(pltpu.VMEM((K, CW), jnp.int32))
    scratch.append(pltpu.VMEM((K, CW), jnp.int32))
    scratch += [pltpu.VMEM((K, CW), jnp.float32)] * ncols
    scratch += [pltpu.SemaphoreType.DMA] * 3

    mesh = plsc.VectorSubcoreMesh(
        core_axis_name="c", subcore_axis_name="s",
        num_cores=NC, num_subcores=NS)

    n_in = 4 + (ncols if use_table else 0)

    def body(*refs):
        src_hbm, dst_hbm, zeros_hbm, ones_hbm = refs[0:4]
        tabs_hbm = refs[4:n_in]
        out_hbm = refs[n_in]
        i = n_in + 1
        accs = refs[i:i + ncols]; i += ncols
        tabs = refs[i:i + ncols] if use_table else ()
        i += ncols if use_table else 0
        src_v = refs[i]; dst_v = refs[i + 1]
        msgs = refs[i + 2:i + 2 + ncols]
        sem_f, sem_g, sem_s = refs[i + 2 + ncols:i + 5 + ncols]

        c = lax.axis_index("c")
        s = lax.axis_index("s")
        sl0 = pl.multiple_of(s * SL, 8)

        for k in range(ncols):
            pltpu.sync_copy(zeros_hbm.at[pl.ds(sl0, SL)],
                            accs[k].at[pl.ds(sl0, SL)])
            if use_table:
                pltpu.sync_copy(tabs_hbm[k].at[pl.ds(sl0, SL)],
                                tabs[k].at[pl.ds(sl0, SL)])
        if not use_table:
            pltpu.sync_copy(ones_hbm, msgs[0])
        plsc.subcore_barrier()

        sup_per_core = n_rows // (K * NC)
        base = c * sup_per_core
        lo = base + (s * sup_per_core) // NS
        hi = base + ((s + 1) * sup_per_core) // NS

        def step(j, carry):
            w = [pltpu.async_copy(dst_hbm.at[j], dst_v, sem_f)]
            if use_table:
                w.append(pltpu.async_copy(src_hbm.at[j], src_v, sem_f))
            for d in w:
                d.wait()
            if use_table:
                w = []
                for u in range(K):
                    for k in range(ncols):
                        w.append(pltpu.async_copy(
                            tabs[k].at[src_v.at[u]], msgs[k].at[u], sem_g))
                for d in w:
                    d.wait()
            w = []
            for u in range(K):
                for k in range(ncols):
                    d = pltpu.make_async_copy(
                        msgs[k].at[u], accs[k].at[dst_v.at[u]], sem_s)
                    d.start(add=True)
                    w.append(d)
            for d in w:
                d.wait()
            return carry

        lax.fori_loop(lo, hi, step, 0)
        plsc.subcore_barrier()

        for k in range(ncols):
            o0 = pl.multiple_of((c * ncols + k) * n_pad + sl0, 8)
            pltpu.sync_copy(accs[k].at[pl.ds(sl0, SL)],
                            out_hbm.at[pl.ds(o0, SL)])

    flat = pl.kernel(
        body,
        out_type=jax.ShapeDtypeStruct((NC * ncols * n_pad,), jnp.float32),
        mesh=mesh,
        scratch_types=scratch,
    )

    def run(*args):
        return flat(*args).reshape(NC, ncols, n_pad)

    return run


def _tc_call(body, n_out, rows):
    return pl.pallas_call(
        body,
        out_shape=[jax.ShapeDtypeStruct((rows, 128), jnp.float32)] * n_out,
    )


def kernel(x, edge_index, W1, b1, W2, b2):
    N = x.shape[0]
    E = edge_index.shape[1]
    fin = W1.shape[0]
    fmid = W1.shape[1]

    n_pad = ((N + 1023) // 1024) * 1024
    if n_pad == N:
        n_pad += 1024
    rows2d = n_pad // 128

    unit = CW * K * NC
    epad = ((E + unit - 1) // unit) * unit
    if epad != E:
        fill = jnp.full((2, epad - E), n_pad - 1, dtype=edge_index.dtype)
        edge_index = jnp.concatenate([edge_index, fill], axis=1)
    R = epad // CW
    src2d = edge_index[0].reshape(R // K, K, CW)
    dst2d = edge_index[1].reshape(R // K, K, CW)

    zeros = jnp.zeros((n_pad,), jnp.float32)
    ones = jnp.ones((K, CW), jnp.float32)

    xcols = jnp.zeros((fin, n_pad), jnp.float32).at[:, :N].set(x.T)

    deg_pass = _build_sc_pass(R, n_pad, 1, use_table=False)
    l1_pass = _build_sc_pass(R, n_pad, fin, use_table=True)
    l2_pass = _build_sc_pass(R, n_pad, 1, use_table=True)

    degp = deg_pass(src2d, dst2d, zeros, ones)

    def tc1(d0, d1, *refs):
        xr = refs[:fin]
        dinv_o = refs[fin]
        xs_o = refs[fin + 1:]
        deg = d0[...] + d1[...] + 1.0
        y = lax.rsqrt(deg)
        dinv = y * (1.5 - 0.5 * deg * y * y)
        dinv_o[...] = dinv
        for k in range(fin):
            xs_o[k][...] = xr[k][...] * dinv

    tc1_out = _tc_call(tc1, 1 + fin, rows2d)(
        degp[0, 0].reshape(rows2d, 128), degp[1, 0].reshape(rows2d, 128),
        *[xcols[k].reshape(rows2d, 128) for k in range(fin)])
    dinv2d = tc1_out[0]
    xs2d = tc1_out[1:]

    xs_flat = [r.reshape(n_pad) for r in xs2d]
    s1 = l1_pass(src2d, dst2d, zeros, ones, *xs_flat)

    def tc2(dinv_r, w1_r, b1_r, w2_r, *refs):
        xs_r = refs[:fin]
        s1_r = refs[fin:fin + 2 * fin]
        gs_o = refs[2 * fin + fin]
        dinv = dinv_r[...]
        agg = [dinv * (s1_r[2 * k][...] + s1_r[2 * k + 1][...] + xs_r[k][...])
               for k in range(fin)]
        g = jnp.zeros_like(dinv)
        for j in range(fmid):
            t = b1_r[0, j]
            for k in range(fin):
                t = t + agg[k] * w1_r[k, j]
            g = g + jnp.maximum(t, 0.0) * w2_r[0, j]
        gs_o[...] = g * dinv

    s1_2d = []
    for k in range(fin):
        s1_2d.append(s1[0, k].reshape(rows2d, 128))
        s1_2d.append(s1[1, k].reshape(rows2d, 128))
    gs2d = _tc_call(tc2, 1, rows2d)(
        dinv2d, W1, b1.reshape(1, fmid), W2.reshape(1, fmid),
        *[xs2d[k] for k in range(fin)], *s1_2d)[0]

    s2 = l2_pass(src2d, dst2d, zeros, ones, gs2d.reshape(n_pad))

    def tc3(dinv_r, gs_r, s20_r, s21_r, b2_r, out_o):
        out_o[...] = dinv_r[...] * (s20_r[...] + s21_r[...] + gs_r[...]) \
            + b2_r[0, 0]

    out2d = _tc_call(tc3, 1, rows2d)(
        dinv2d, gs2d, s2[0, 0].reshape(rows2d, 128),
        s2[1, 0].reshape(rows2d, 128), b2.reshape(1, 1))[0]

    return out2d.reshape(n_pad)[:N].reshape(N, 1)
